# baseline (speedup 1.0000x reference)
import functools

import jax
import jax.numpy as jnp
from jax import lax
from jax.experimental import pallas as pl
from jax.experimental.pallas import tpu as pltpu
from jax.experimental.pallas import tpu_sc as plsc

N = 10000
E = 320000
D = 128

NC = 2
NS = 16
NW = NC * NS
EPW = E // NW
CHUNK = 50
NCHUNK = EPW // CHUNK
NROW = 4
NIDX = 8
DTILES = 10
DR = N // DTILES


def _sc_segment_sum(x, srcs, dsts, zeros):
    mesh = plsc.VectorSubcoreMesh(core_axis_name="c", subcore_axis_name="s")

    @functools.partial(
        pl.kernel,
        mesh=mesh,
        out_type=jax.ShapeDtypeStruct((NC, N, D), jnp.float32),
        scratch_types=[
            pltpu.VMEM((NIDX, CHUNK), jnp.int32),
            pltpu.VMEM((NIDX, CHUNK), jnp.int32),
            pltpu.VMEM((NROW, CHUNK, D), jnp.float32),
            pltpu.VMEM_SHARED((N, D), jnp.float32),
            [pltpu.SemaphoreType.DMA] * NIDX,
            [pltpu.SemaphoreType.DMA] * NROW,
            [pltpu.SemaphoreType.DMA] * NROW,
        ],
    )
    def body(x_hbm, src_hbm, dst_hbm, zero_hbm, out_hbm, src_v, dst_v,
             rows_v, acc_sh, isems, gsems, ssems):
        c = lax.axis_index("c")
        s = lax.axis_index("s")
        wid = s * NC + c

        def idx_load(cj, j):
            q = j % NIDX
            pltpu.make_async_copy(
                src_hbm.at[wid, cj], src_v.at[q], isems[q]).start()
            pltpu.make_async_copy(
                dst_hbm.at[wid, cj], dst_v.at[q], isems[q]).start()

        def idx_wait(j):
            q = j % NIDX
            pltpu.make_async_copy(
                src_hbm.at[wid, 0], src_v.at[q], isems[q]).wait()
            pltpu.make_async_copy(
                dst_hbm.at[wid, 0], dst_v.at[q], isems[q]).wait()

        def gather_start(j):
            pltpu.make_async_copy(
                x_hbm.at[src_v.at[j % NIDX]], rows_v.at[j % NROW],
                gsems[j % NROW]).start()

        def gather_wait(j):
            pltpu.make_async_copy(
                x_hbm.at[src_v.at[j % NIDX]], rows_v.at[j % NROW],
                gsems[j % NROW]).wait()

        def scat_start(j):
            pltpu.make_async_copy(
                rows_v.at[j % NROW], acc_sh.at[dst_v.at[j % NIDX]],
                ssems[j % NROW]).start(add=True)

        def scat_wait(j):
            pltpu.make_async_copy(
                rows_v.at[j % NROW], acc_sh.at[dst_v.at[j % NIDX]],
                ssems[j % NROW]).wait()

        for cj in range(NIDX - 2):
            idx_load(cj, cj)

        @pl.when(s < DTILES)
        def _():
            pltpu.sync_copy(zero_hbm.at[pl.ds(s * DR, DR)],
                            acc_sh.at[pl.ds(s * DR, DR)])

        idx_wait(0)
        gather_start(0)
        idx_wait(1)
        gather_start(1)

        plsc.subcore_barrier()

        def step(ci, j, lo_ok, hi6_ok, hi2_ok):
            gather_wait(j)
            scat_start(j)
            if lo_ok:
                scat_wait(j - 2)
            if hi6_ok:
                idx_load(ci + 6, j + 6)
            if hi2_ok:
                idx_wait(j + 2)
                gather_start(j + 2)

        for j in range(8):
            step(j, j, j >= 2, True, True)

        def cycle(k, carry):
            for j in range(8):
                step(8 * k + j, j, True, True, True)
            return carry
        lax.fori_loop(1, NCHUNK // 8 - 1, cycle, 0)

        for j in range(8):
            ci = NCHUNK - 8 + j
            step(ci, j, True, ci + 6 < NCHUNK, ci + 2 < NCHUNK)

        scat_wait(NCHUNK - 2)
        scat_wait(NCHUNK - 1)

        plsc.subcore_barrier()

        @pl.when(s < DTILES)
        def _():
            pltpu.sync_copy(acc_sh.at[pl.ds(s * DR, DR)],
                            out_hbm.at[c, pl.ds(s * DR, DR)])

    return body(x, srcs, dsts, zeros)


BLK = 1000


def _tc_combine(part, x, weight, weight_self):
    def body(p_ref, x_ref, w_ref, ws_ref, o_ref):
        agg = p_ref[0] + p_ref[1]
        o_ref[...] = (
            jnp.dot(agg, w_ref[...], preferred_element_type=jnp.float32)
            + jnp.dot(x_ref[...], ws_ref[...], preferred_element_type=jnp.float32)
        )

    return pl.pallas_call(
        body,
        grid=(N // BLK,),
        in_specs=[
            pl.BlockSpec((NC, BLK, D), lambda i: (0, i, 0)),
            pl.BlockSpec((BLK, D), lambda i: (i, 0)),
            pl.BlockSpec((D, D), lambda i: (0, 0)),
            pl.BlockSpec((D, D), lambda i: (0, 0)),
        ],
        out_specs=pl.BlockSpec((BLK, D), lambda i: (i, 0)),
        out_shape=jax.ShapeDtypeStruct((N, D), jnp.float32),
    )(part, x, weight, weight_self)


def kernel(x, edge_index, weight, weight_self):
    srcs = edge_index[0].reshape(NW, NCHUNK, CHUNK)
    dsts = edge_index[1].reshape(NW, NCHUNK, CHUNK)
    zeros = jnp.zeros((N, D), jnp.float32)
    part = _sc_segment_sum(x, srcs, dsts, zeros)
    return _tc_combine(part, x, weight, weight_self)


# speedup vs baseline: 1.2328x; 1.2328x over previous
import functools

import jax
import jax.numpy as jnp
from jax import lax
from jax.experimental import pallas as pl
from jax.experimental.pallas import tpu as pltpu
from jax.experimental.pallas import tpu_sc as plsc

N = 10000
E = 320000
D = 128

NC = 2
NS = 16
NW = NC * NS
EPW = E // NW
CHUNK = 80
NCHUNK = EPW // CHUNK
DTILES = 10
DR = N // DTILES
ZR = 200


def _sc_segment_sum(x, edge_flat):
    mesh = plsc.VectorSubcoreMesh(core_axis_name="c", subcore_axis_name="s")

    @functools.partial(
        pl.kernel,
        mesh=mesh,
        out_type=jax.ShapeDtypeStruct((NC, N, D), jnp.float32),
        scratch_types=[
            pltpu.VMEM((4, CHUNK), jnp.int32),
            pltpu.VMEM((4, CHUNK), jnp.int32),
            pltpu.VMEM((CHUNK, D), jnp.float32),
            pltpu.VMEM((CHUNK, D), jnp.float32),
            pltpu.VMEM((ZR, D), jnp.float32),
            pltpu.VMEM_SHARED((N, D), jnp.float32),
            pltpu.SemaphoreType.DMA,
            pltpu.SemaphoreType.DMA,
            pltpu.SemaphoreType.DMA,
            pltpu.SemaphoreType.DMA,
            pltpu.SemaphoreType.DMA,
            pltpu.SemaphoreType.DMA,
        ],
    )
    def body(x_hbm, ei_hbm, out_hbm, src_v, dst_v, rows0, rows1, zero_v,
             acc_sh, is0, is1, is2, is3, gsem0, gsem1):
        c = lax.axis_index("c")
        s = lax.axis_index("s")
        wid = s * NC + c
        base_w = wid * EPW

        bufs = (rows0, rows1)
        gsems = (gsem0, gsem1)
        isems = (is0, is1, is2, is3)

        def idx_load(ci, slot):
            off = base_w + ci * CHUNK
            pltpu.make_async_copy(
                ei_hbm.at[pl.ds(off, CHUNK)], src_v.at[slot],
                isems[slot]).start()
            pltpu.make_async_copy(
                ei_hbm.at[pl.ds(E + off, CHUNK)], dst_v.at[slot],
                isems[slot]).start()

        def idx_wait(slot):
            pltpu.make_async_copy(
                ei_hbm.at[pl.ds(0, CHUNK)], src_v.at[slot],
                isems[slot]).wait()
            pltpu.make_async_copy(
                ei_hbm.at[pl.ds(0, CHUNK)], dst_v.at[slot],
                isems[slot]).wait()

        def gather_start(b, slot):
            pltpu.make_async_copy(
                x_hbm.at[src_v.at[slot]], bufs[b], gsems[b]).start()

        def gather_wait(b, slot):
            pltpu.make_async_copy(
                x_hbm.at[src_v.at[slot]], bufs[b], gsems[b]).wait()

        for ci in range(4):
            idx_load(ci, ci)

        @pl.when(s < DTILES)
        def _():
            def zrow(i, carry):
                def zcol(j, carry2):
                    zero_v[i, pl.ds(j * 16, 16)] = jnp.zeros((16,),
                                                             jnp.float32)
                    return carry2
                return lax.fori_loop(0, D // 16, zcol, carry)
            lax.fori_loop(0, ZR, zrow, 0)
            for z in range(DR // ZR):
                pltpu.sync_copy(zero_v, acc_sh.at[pl.ds(s * DR + z * ZR, ZR)])

        idx_wait(0)
        gather_start(0, 0)
        idx_wait(1)
        gather_start(1, 1)

        plsc.subcore_barrier()

        def step(ci, b, slot):
            gather_wait(b, slot)
            pltpu.sync_copy(bufs[b], acc_sh.at[dst_v.at[slot]], add=True)

            @pl.when(ci + 4 < NCHUNK)
            def _():
                idx_load(ci + 4, slot)

            @pl.when(ci + 2 < NCHUNK)
            def _():
                nslot = (slot + 2) % 4
                idx_wait(nslot)
                gather_start(b, nslot)

        def quad(k, carry):
            ci = 4 * k
            step(ci, 0, 0)
            step(ci + 1, 1, 1)
            step(ci + 2, 0, 2)
            step(ci + 3, 1, 3)
            return carry
        lax.fori_loop(0, NCHUNK // 4, quad, 0)
        if NCHUNK % 4:
            for j in range(NCHUNK % 4):
                ci = (NCHUNK // 4) * 4 + j
                step(ci, j % 2, j)

        plsc.subcore_barrier()

        @pl.when(s < DTILES)
        def _():
            pltpu.sync_copy(acc_sh.at[pl.ds(s * DR, DR)],
                            out_hbm.at[c, pl.ds(s * DR, DR)])

    return body(x, edge_flat)


BLK = 1000


def _tc_combine(part, x, weight, weight_self):
    def body(p_ref, x_ref, w_ref, ws_ref, o_ref):
        agg = p_ref[0] + p_ref[1]
        o_ref[...] = (
            jnp.dot(agg, w_ref[...], preferred_element_type=jnp.float32)
            + jnp.dot(x_ref[...], ws_ref[...], preferred_element_type=jnp.float32)
        )

    return pl.pallas_call(
        body,
        grid=(N // BLK,),
        in_specs=[
            pl.BlockSpec((NC, BLK, D), lambda i: (0, i, 0)),
            pl.BlockSpec((BLK, D), lambda i: (i, 0)),
            pl.BlockSpec((D, D), lambda i: (0, 0)),
            pl.BlockSpec((D, D), lambda i: (0, 0)),
        ],
        out_specs=pl.BlockSpec((BLK, D), lambda i: (i, 0)),
        out_shape=jax.ShapeDtypeStruct((N, D), jnp.float32),
    )(part, x, weight, weight_self)


def kernel(x, edge_index, weight, weight_self):
    part = _sc_segment_sum(x, edge_index.reshape(-1))
    return _tc_combine(part, x, weight, weight_self)


# speedup vs baseline: 1.2530x; 1.0164x over previous
import functools

import jax
import jax.numpy as jnp
from jax import lax
from jax.experimental import pallas as pl
from jax.experimental.pallas import tpu as pltpu
from jax.experimental.pallas import tpu_sc as plsc

N = 10000
E = 320000
D = 128

NC = 2
NS = 16
NW = NC * NS
EPW = E // NW
CHUNK = 80
NCHUNK = EPW // CHUNK
DTILES = 10
DR = N // DTILES
ZR = 40


def _sc_segment_sum(x, edge_flat):
    mesh = plsc.VectorSubcoreMesh(core_axis_name="c", subcore_axis_name="s")

    @functools.partial(
        pl.kernel,
        mesh=mesh,
        out_type=jax.ShapeDtypeStruct((NC, N, D), jnp.float32),
        scratch_types=[
            pltpu.VMEM((4, CHUNK), jnp.int32),
            pltpu.VMEM((4, CHUNK), jnp.int32),
            pltpu.VMEM((4, CHUNK), jnp.int32),
            pltpu.VMEM((4, CHUNK, D), jnp.float32),
            pltpu.VMEM((ZR, D), jnp.float32),
            pltpu.VMEM_SHARED((N, D), jnp.float32),
            [pltpu.SemaphoreType.DMA] * 4,
            [pltpu.SemaphoreType.DMA] * 4,
            [pltpu.SemaphoreType.DMA] * 4,
        ],
    )
    def body(x_hbm, ei_hbm, out_hbm, src_v, dst_v, sidx_v, rows_v, zero_v,
             acc_sh, isems, gsems, ssems):
        c = lax.axis_index("c")
        s = lax.axis_index("s")
        wid = s * NC + c
        base_w = wid * EPW

        def idx_load(ci, j):
            q = j % 4
            off = base_w + ci * CHUNK
            pltpu.make_async_copy(
                ei_hbm.at[pl.ds(off, CHUNK)], src_v.at[q],
                isems[q]).start()
            pltpu.make_async_copy(
                ei_hbm.at[pl.ds(E + off, CHUNK)], dst_v.at[q],
                isems[q]).start()

        def idx_wait(j):
            q = j % 4
            pltpu.make_async_copy(
                ei_hbm.at[pl.ds(0, CHUNK)], src_v.at[q], isems[q]).wait()
            pltpu.make_async_copy(
                ei_hbm.at[pl.ds(0, CHUNK)], dst_v.at[q], isems[q]).wait()

        def gather_start(j):
            q = j % 4
            pltpu.make_async_copy(
                x_hbm.at[src_v.at[q]], rows_v.at[q], gsems[q]).start()

        def gather_wait(j):
            q = j % 4
            pltpu.make_async_copy(
                x_hbm.at[src_v.at[q]], rows_v.at[q], gsems[q]).wait()

        def scat_start(j):
            q = j % 4
            def cp(i, carry):
                sidx_v[q, pl.ds(i * 16, 16)] = dst_v[q, pl.ds(i * 16, 16)]
                return carry
            lax.fori_loop(0, CHUNK // 16, cp, 0)
            pltpu.make_async_copy(
                rows_v.at[q], acc_sh.at[sidx_v.at[q]],
                ssems[q]).start(add=True)

        def scat_wait(j):
            q = j % 4
            pltpu.make_async_copy(
                rows_v.at[q], acc_sh.at[sidx_v.at[q]], ssems[q]).wait()

        for cj in range(4):
            idx_load(cj, cj)

        @pl.when(s < DTILES)
        def _():
            def zrow(i, carry):
                def zcol(jj, carry2):
                    zero_v[i, pl.ds(jj * 16, 16)] = jnp.zeros((16,),
                                                              jnp.float32)
                    return carry2
                return lax.fori_loop(0, D // 16, zcol, carry)
            lax.fori_loop(0, ZR, zrow, 0)
            for z in range(DR // ZR):
                pltpu.sync_copy(zero_v, acc_sh.at[pl.ds(s * DR + z * ZR, ZR)])

        idx_wait(0)
        gather_start(0)
        idx_wait(1)
        gather_start(1)

        plsc.subcore_barrier()

        def step(ci, j, lo_ok):
            gather_wait(j)
            scat_start(j)
            if lo_ok:
                scat_wait(j + 2)

            @pl.when(ci + 4 < NCHUNK)
            def _():
                idx_load(ci + 4, j)

            @pl.when(ci + 2 < NCHUNK)
            def _():
                idx_wait(j + 2)
                gather_start(j + 2)

        for j in range(4):
            step(j, j, j >= 2)

        def quad(k, carry):
            for j in range(4):
                step(4 * k + j, j, True)
            return carry
        lax.fori_loop(1, NCHUNK // 4, quad, 0)

        step(NCHUNK - 1, 0, True)

        scat_wait(NCHUNK - 2)
        scat_wait(NCHUNK - 1)

        plsc.subcore_barrier()

        @pl.when(s < DTILES)
        def _():
            pltpu.sync_copy(acc_sh.at[pl.ds(s * DR, DR)],
                            out_hbm.at[c, pl.ds(s * DR, DR)])

    return body(x, edge_flat)


BLK = 1000


def _tc_combine(part, x, weight, weight_self):
    def body(p_ref, x_ref, w_ref, ws_ref, o_ref):
        agg = p_ref[0] + p_ref[1]
        o_ref[...] = (
            jnp.dot(agg, w_ref[...], preferred_element_type=jnp.float32)
            + jnp.dot(x_ref[...], ws_ref[...], preferred_element_type=jnp.float32)
        )

    return pl.pallas_call(
        body,
        grid=(N // BLK,),
        in_specs=[
            pl.BlockSpec((NC, BLK, D), lambda i: (0, i, 0)),
            pl.BlockSpec((BLK, D), lambda i: (i, 0)),
            pl.BlockSpec((D, D), lambda i: (0, 0)),
            pl.BlockSpec((D, D), lambda i: (0, 0)),
        ],
        out_specs=pl.BlockSpec((BLK, D), lambda i: (i, 0)),
        out_shape=jax.ShapeDtypeStruct((N, D), jnp.float32),
    )(part, x, weight, weight_self)


def kernel(x, edge_index, weight, weight_self):
    part = _sc_segment_sum(x, edge_index.reshape(-1))
    return _tc_combine(part, x, weight, weight_self)


# speedup vs baseline: 1.2648x; 1.0094x over previous
import functools

import jax
import jax.numpy as jnp
from jax import lax
from jax.experimental import pallas as pl
from jax.experimental.pallas import tpu as pltpu
from jax.experimental.pallas import tpu_sc as plsc

N = 10000
E = 320000
D = 128

NC = 2
NS = 16
NW = NC * NS
EPW = E // NW
CHUNK = 80
NCHUNK = EPW // CHUNK
DTILES = 10
DR = N // DTILES
ZR = 40


def _sc_segment_sum(x, edge_flat):
    mesh = plsc.VectorSubcoreMesh(core_axis_name="c", subcore_axis_name="s")

    @functools.partial(
        pl.kernel,
        mesh=mesh,
        out_type=jax.ShapeDtypeStruct((NC, N, D), jnp.float32),
        scratch_types=[
            pltpu.VMEM((4, CHUNK), jnp.int32),
            pltpu.VMEM((4, CHUNK), jnp.int32),
            pltpu.VMEM((4, CHUNK), jnp.int32),
            pltpu.VMEM((4, CHUNK, D), jnp.float32),
            pltpu.VMEM((ZR, D), jnp.float32),
            pltpu.VMEM_SHARED((N, D), jnp.float32),
            [pltpu.SemaphoreType.DMA] * 4,
            [pltpu.SemaphoreType.DMA] * 4,
            [pltpu.SemaphoreType.DMA] * 4,
            pltpu.SemaphoreType.DMA,
        ],
    )
    def body(x_hbm, ei_hbm, out_hbm, src_v, dst_v, sidx_v, rows_v, zero_v,
             acc_sh, isems, gsems, ssems, zsem):
        c = lax.axis_index("c")
        s = lax.axis_index("s")
        wid = s * NC + c
        base_w = wid * EPW

        def idx_load(ci, j):
            q = j % 4
            off = base_w + ci * CHUNK
            pltpu.make_async_copy(
                ei_hbm.at[pl.ds(off, CHUNK)], src_v.at[q],
                isems[q]).start()
            pltpu.make_async_copy(
                ei_hbm.at[pl.ds(E + off, CHUNK)], dst_v.at[q],
                isems[q]).start()

        def idx_wait(j):
            q = j % 4
            pltpu.make_async_copy(
                ei_hbm.at[pl.ds(0, CHUNK)], src_v.at[q], isems[q]).wait()
            pltpu.make_async_copy(
                ei_hbm.at[pl.ds(0, CHUNK)], dst_v.at[q], isems[q]).wait()

        def gather_start(j):
            q = j % 4
            pltpu.make_async_copy(
                x_hbm.at[src_v.at[q]], rows_v.at[q], gsems[q]).start()

        def gather_wait(j):
            q = j % 4
            pltpu.make_async_copy(
                x_hbm.at[src_v.at[q]], rows_v.at[q], gsems[q]).wait()

        def scat_start(j):
            q = j % 4
            def cp(i, carry):
                sidx_v[q, pl.ds(i * 16, 16)] = dst_v[q, pl.ds(i * 16, 16)]
                return carry
            lax.fori_loop(0, CHUNK // 16, cp, 0)
            pltpu.make_async_copy(
                rows_v.at[q], acc_sh.at[sidx_v.at[q]],
                ssems[q]).start(add=True)

        def scat_wait(j):
            q = j % 4
            pltpu.make_async_copy(
                rows_v.at[q], acc_sh.at[sidx_v.at[q]], ssems[q]).wait()

        for cj in range(4):
            idx_load(cj, cj)

        @pl.when(s < DTILES)
        def _():
            def zrow(i, carry):
                def zcol(jj, carry2):
                    zero_v[i, pl.ds(jj * 16, 16)] = jnp.zeros((16,),
                                                              jnp.float32)
                    return carry2
                return lax.fori_loop(0, D // 16, zcol, carry)
            lax.fori_loop(0, ZR, zrow, 0)
            for z in range(DR // ZR):
                pltpu.make_async_copy(
                    zero_v, acc_sh.at[pl.ds(s * DR + z * ZR, ZR)],
                    zsem).start()
            for z in range(DR // ZR):
                pltpu.make_async_copy(
                    zero_v, acc_sh.at[pl.ds(s * DR + z * ZR, ZR)],
                    zsem).wait()

        idx_wait(0)
        gather_start(0)
        idx_wait(1)
        gather_start(1)

        plsc.subcore_barrier()

        def step(ci, j, lo_ok):
            gather_wait(j)
            scat_start(j)
            if lo_ok:
                scat_wait(j + 2)

            @pl.when(ci + 4 < NCHUNK)
            def _():
                idx_load(ci + 4, j)

            @pl.when(ci + 2 < NCHUNK)
            def _():
                idx_wait(j + 2)
                gather_start(j + 2)

        for j in range(4):
            step(j, j, j >= 2)

        def quad(k, carry):
            for j in range(4):
                step(4 * k + j, j, True)
            return carry
        lax.fori_loop(1, NCHUNK // 4, quad, 0)

        step(NCHUNK - 1, 0, True)

        scat_wait(NCHUNK - 2)
        scat_wait(NCHUNK - 1)

        plsc.subcore_barrier()

        @pl.when(s < DTILES)
        def _():
            pltpu.sync_copy(acc_sh.at[pl.ds(s * DR, DR)],
                            out_hbm.at[c, pl.ds(s * DR, DR)])

    return body(x, edge_flat)


BLK = 1000


def _tc_combine(part, x, weight, weight_self):
    def body(p_ref, x_ref, w_ref, ws_ref, o_ref):
        agg = p_ref[0] + p_ref[1]
        o_ref[...] = (
            jnp.dot(agg, w_ref[...], preferred_element_type=jnp.float32)
            + jnp.dot(x_ref[...], ws_ref[...], preferred_element_type=jnp.float32)
        )

    return pl.pallas_call(
        body,
        grid=(N // BLK,),
        in_specs=[
            pl.BlockSpec((NC, BLK, D), lambda i: (0, i, 0)),
            pl.BlockSpec((BLK, D), lambda i: (i, 0)),
            pl.BlockSpec((D, D), lambda i: (0, 0)),
            pl.BlockSpec((D, D), lambda i: (0, 0)),
        ],
        out_specs=pl.BlockSpec((BLK, D), lambda i: (i, 0)),
        out_shape=jax.ShapeDtypeStruct((N, D), jnp.float32),
    )(part, x, weight, weight_self)


def kernel(x, edge_index, weight, weight_self):
    part = _sc_segment_sum(x, edge_index.reshape(-1))
    return _tc_combine(part, x, weight, weight_self)


# speedup vs baseline: 1.2682x; 1.0027x over previous
import functools

import jax
import jax.numpy as jnp
from jax import lax
from jax.experimental import pallas as pl
from jax.experimental.pallas import tpu as pltpu
from jax.experimental.pallas import tpu_sc as plsc

N = 10000
E = 320000
D = 128

NC = 2
NS = 16
NW = NC * NS
EPW = E // NW
CHUNK = 80
NCHUNK = EPW // CHUNK
DTILES = 10
DR = N // DTILES
ZR = 40


def _sc_segment_sum(x, edge_flat):
    mesh = plsc.VectorSubcoreMesh(core_axis_name="c", subcore_axis_name="s")

    @functools.partial(
        pl.kernel,
        mesh=mesh,
        out_type=jax.ShapeDtypeStruct((NC, N, D), jnp.float32),
        scratch_types=[
            pltpu.VMEM((4, CHUNK), jnp.int32),
            pltpu.VMEM((4, CHUNK), jnp.int32),
            pltpu.VMEM((4, CHUNK), jnp.int32),
            pltpu.VMEM((4, CHUNK, D), jnp.float32),
            pltpu.VMEM((ZR, D), jnp.float32),
            pltpu.VMEM_SHARED((N, D), jnp.float32),
            [pltpu.SemaphoreType.DMA] * 4,
            [pltpu.SemaphoreType.DMA] * 4,
            [pltpu.SemaphoreType.DMA] * 4,
            pltpu.SemaphoreType.DMA,
        ],
    )
    def body(x_hbm, ei_hbm, out_hbm, src_v, dst_v, sidx_v, rows_v, zero_v,
             acc_sh, isems, gsems, ssems, zsem):
        c = lax.axis_index("c")
        s = lax.axis_index("s")
        wid = s * NC + c
        base_w = wid * EPW

        def idx_load(ci, j):
            q = j % 4
            off = base_w + ci * CHUNK
            pltpu.make_async_copy(
                ei_hbm.at[pl.ds(off, CHUNK)], src_v.at[q],
                isems[q]).start()
            pltpu.make_async_copy(
                ei_hbm.at[pl.ds(E + off, CHUNK)], dst_v.at[q],
                isems[q]).start()

        def idx_wait(j):
            q = j % 4
            pltpu.make_async_copy(
                ei_hbm.at[pl.ds(0, CHUNK)], src_v.at[q], isems[q]).wait()
            pltpu.make_async_copy(
                ei_hbm.at[pl.ds(0, CHUNK)], dst_v.at[q], isems[q]).wait()

        def gather_start(j):
            q = j % 4
            pltpu.make_async_copy(
                x_hbm.at[src_v.at[q]], rows_v.at[q], gsems[q]).start()

        def gather_wait(j):
            q = j % 4
            pltpu.make_async_copy(
                x_hbm.at[src_v.at[q]], rows_v.at[q], gsems[q]).wait()

        def scat_start(j):
            q = j % 4
            def cp(i, carry):
                sidx_v[q, pl.ds(i * 16, 16)] = dst_v[q, pl.ds(i * 16, 16)]
                return carry
            lax.fori_loop(0, CHUNK // 16, cp, 0)
            pltpu.make_async_copy(
                rows_v.at[q], acc_sh.at[sidx_v.at[q]],
                ssems[q]).start(add=True)

        def scat_wait(j):
            q = j % 4
            pltpu.make_async_copy(
                rows_v.at[q], acc_sh.at[sidx_v.at[q]], ssems[q]).wait()

        for cj in range(4):
            idx_load(cj, cj)

        @pl.when(s < DTILES)
        def _():
            def zrow(i, carry):
                def zcol(jj, carry2):
                    zero_v[i, pl.ds(jj * 16, 16)] = jnp.zeros((16,),
                                                              jnp.float32)
                    return carry2
                return lax.fori_loop(0, D // 16, zcol, carry)
            lax.fori_loop(0, ZR, zrow, 0)
            for z in range(DR // ZR):
                pltpu.make_async_copy(
                    zero_v, acc_sh.at[pl.ds(s * DR + z * ZR, ZR)],
                    zsem).start()
            for z in range(DR // ZR):
                pltpu.make_async_copy(
                    zero_v, acc_sh.at[pl.ds(s * DR + z * ZR, ZR)],
                    zsem).wait()

        idx_wait(0)
        gather_start(0)
        idx_wait(1)
        gather_start(1)

        plsc.subcore_barrier()

        def step(ci, j, lo_ok):
            gather_wait(j)
            scat_start(j)
            if lo_ok:
                scat_wait(j + 2)

            @pl.when(ci + 4 < NCHUNK)
            def _():
                idx_load(ci + 4, j)

            @pl.when(ci + 2 < NCHUNK)
            def _():
                idx_wait(j + 2)
                gather_start(j + 2)

        for j in range(4):
            step(j, j, j >= 2)

        def quad(k, carry):
            for j in range(4):
                step(4 * k + j, j, True)
            return carry
        lax.fori_loop(1, NCHUNK // 4, quad, 0)

        step(NCHUNK - 1, 0, True)

        scat_wait(NCHUNK - 2)
        scat_wait(NCHUNK - 1)

        plsc.subcore_barrier()

        @pl.when(s < DTILES)
        def _():
            pltpu.sync_copy(acc_sh.at[pl.ds(s * DR, DR)],
                            out_hbm.at[c, pl.ds(s * DR, DR)])

    return body(x, edge_flat)


BLK = 1000


def _tc_self(x, weight_self):
    def body(x_ref, ws_ref, o_ref):
        o_ref[...] = jnp.dot(x_ref[...], ws_ref[...],
                             preferred_element_type=jnp.float32)

    return pl.pallas_call(
        body,
        grid=(N // BLK,),
        in_specs=[
            pl.BlockSpec((BLK, D), lambda i: (i, 0)),
            pl.BlockSpec((D, D), lambda i: (0, 0)),
        ],
        out_specs=pl.BlockSpec((BLK, D), lambda i: (i, 0)),
        out_shape=jax.ShapeDtypeStruct((N, D), jnp.float32),
    )(x, weight_self)


def _tc_combine(part, xs, weight):
    def body(p_ref, xs_ref, w_ref, o_ref):
        agg = p_ref[0] + p_ref[1]
        o_ref[...] = (
            jnp.dot(agg, w_ref[...], preferred_element_type=jnp.float32)
            + xs_ref[...]
        )

    return pl.pallas_call(
        body,
        grid=(N // BLK,),
        in_specs=[
            pl.BlockSpec((NC, BLK, D), lambda i: (0, i, 0)),
            pl.BlockSpec((BLK, D), lambda i: (i, 0)),
            pl.BlockSpec((D, D), lambda i: (0, 0)),
        ],
        out_specs=pl.BlockSpec((BLK, D), lambda i: (i, 0)),
        out_shape=jax.ShapeDtypeStruct((N, D), jnp.float32),
    )(part, xs, weight)


def kernel(x, edge_index, weight, weight_self):
    xs = _tc_self(x, weight_self)
    part = _sc_segment_sum(x, edge_index.reshape(-1))
    return _tc_combine(part, xs, weight)
